# baseline (speedup 1.0000x reference)
import jax
import jax.numpy as jnp
from jax.experimental import pallas as pl

_B, _N, _M = 16, 2048, 64
_LANES = 128
_NR = _N * _M // _LANES
_R = 512
_NC = _NR // _R


def _body(dist_ref, pt_ref, types_ref, coef_ref, out_ref, epa_ref):
    b = pl.program_id(0)
    j = pl.program_id(1)
    d = dist_ref[0]
    pt = pt_ref[0]
    is1 = pt == 1
    is2 = pt == 2

    def sel(i):
        return jnp.where(is1, coef_ref[i, 1],
                         jnp.where(is2, coef_ref[i, 2], coef_ref[i, 0]))

    p0 = sel(0)
    p1 = sel(1)
    q0 = sel(2)
    q1 = sel(3)
    ga = sel(4)
    de = sel(5)

    x = jnp.clip(de * d - ga, 0.0, 1.0)
    x3 = x * x * x
    fc = 1.0 - x3 * (x * (6.0 * x - 15.0) + 10.0)

    half_phi = jnp.exp2(p0 - p1 * d) * fc
    rho_e = jnp.exp2(q0 - q1 * d) * fc

    half_phi_sum = jnp.sum(half_phi)

    lane = jax.lax.broadcasted_iota(jnp.int32, (_R, _LANES), 1)
    low = lane < _M
    s0 = jnp.sum(jnp.where(low, rho_e, 0.0), axis=1, keepdims=True) + 1e-12
    s1 = jnp.sum(rho_e, axis=1, keepdims=True) + 2e-12 - s0
    emb_sum = jnp.sum(s0 * jax.lax.rsqrt(s0) + s1 * jax.lax.rsqrt(s1))

    e = jnp.reshape(half_phi_sum - emb_sum, (1, 1))

    @pl.when(j == 0)
    def _init():
        tt = types_ref[pl.ds(b, 1), :]
        off_sum = jnp.sum(jnp.where(tt == 1, coef_ref[6, 1], coef_ref[6, 0]))
        out_ref[pl.ds(b, 1), :] = e + off_sum

    @pl.when(j != 0)
    def _acc():
        out_ref[pl.ds(b, 1), :] += e

    @pl.when(j == _NC - 1)
    def _fin():
        epa_ref[pl.ds(b, 1), :] = out_ref[pl.ds(b, 1), :] * (1.0 / _N)


def kernel(types, pair_types, distances, A, xi, p, q, r0, offset, cut_a, cut_b):
    dist = distances.reshape(_B, _NR, _LANES)
    pt = pair_types.reshape(_B, _NR, _LANES)

    inv_ln2 = 1.4426950408889634
    inv_ba = 1.0 / (cut_b - cut_a)
    coef = jnp.concatenate([
        jnp.stack([
            jnp.log2(0.5 * A) + p * inv_ln2,
            (p / r0) * inv_ln2,
            2.0 * jnp.log2(xi) + 2.0 * q * inv_ln2,
            (2.0 * q / r0) * inv_ln2,
            cut_a * inv_ba,
            inv_ba,
        ]),
        jnp.pad(offset, (0, 1)).reshape(1, 3),
    ])

    energy, energy_per_atom = pl.pallas_call(
        _body,
        grid=(_B, _NC),
        in_specs=[
            pl.BlockSpec((1, _R, _LANES), lambda b, j: (b, j, 0)),
            pl.BlockSpec((1, _R, _LANES), lambda b, j: (b, j, 0)),
            pl.BlockSpec((_B, _N), lambda b, j: (0, 0)),
            pl.BlockSpec((7, 3), lambda b, j: (0, 0)),
        ],
        out_specs=[
            pl.BlockSpec((_B, 1), lambda b, j: (0, 0)),
            pl.BlockSpec((_B, 1), lambda b, j: (0, 0)),
        ],
        out_shape=[
            jax.ShapeDtypeStruct((_B, 1), jnp.float32),
            jax.ShapeDtypeStruct((_B, 1), jnp.float32),
        ],
    )(dist, pt, types, coef)

    return (energy, energy_per_atom)


# speedup vs baseline: 2.9883x; 2.9883x over previous
import jax
import jax.numpy as jnp
from jax.experimental import pallas as pl

_B, _N, _M = 16, 2048, 64


def _body(dist_ref, pt_ref, types_ref, coef_ref, out_ref, epa_ref):
    b = pl.program_id(0)
    d = dist_ref[0]
    pt = pt_ref[0]
    is1 = pt == 1
    is2 = pt == 2

    def sel(i):
        return jnp.where(is1, coef_ref[i, 1],
                         jnp.where(is2, coef_ref[i, 2], coef_ref[i, 0]))

    p0 = sel(0)
    p1 = sel(1)
    q0 = sel(2)
    q1 = sel(3)
    ga = sel(4)
    de = sel(5)

    x = jnp.clip(de * d - ga, 0.0, 1.0)
    x3 = x * x * x
    fc = 1.0 - x3 * (x * (6.0 * x - 15.0) + 10.0)

    half_phi = jnp.exp2(p0 - p1 * d) * fc
    rho_e = jnp.exp2(q0 - q1 * d) * fc

    half_phi_sum = jnp.sum(half_phi)
    s = jnp.sum(rho_e, axis=0, keepdims=True) + 1e-12
    emb_sum = jnp.sum(s * jax.lax.rsqrt(s))

    tt = types_ref[pl.ds(b, 1), :]
    off_sum = jnp.sum(jnp.where(tt == 1, coef_ref[6, 1], coef_ref[6, 0]))

    e = jnp.reshape(half_phi_sum - emb_sum + off_sum, (1, 1))
    out_ref[pl.ds(b, 1), :] = e
    epa_ref[pl.ds(b, 1), :] = e * (1.0 / _N)


def kernel(types, pair_types, distances, A, xi, p, q, r0, offset, cut_a, cut_b):
    dist_t = distances.transpose(0, 2, 1)
    pt_t = pair_types.transpose(0, 2, 1)

    inv_ln2 = 1.4426950408889634
    inv_ba = 1.0 / (cut_b - cut_a)
    coef = jnp.concatenate([
        jnp.stack([
            jnp.log2(0.5 * A) + p * inv_ln2,
            (p / r0) * inv_ln2,
            2.0 * jnp.log2(xi) + 2.0 * q * inv_ln2,
            (2.0 * q / r0) * inv_ln2,
            cut_a * inv_ba,
            inv_ba,
        ]),
        jnp.pad(offset, (0, 1)).reshape(1, 3),
    ])

    energy, energy_per_atom = pl.pallas_call(
        _body,
        grid=(_B,),
        in_specs=[
            pl.BlockSpec((1, _M, _N), lambda b: (b, 0, 0)),
            pl.BlockSpec((1, _M, _N), lambda b: (b, 0, 0)),
            pl.BlockSpec((_B, _N), lambda b: (0, 0)),
            pl.BlockSpec((7, 3), lambda b: (0, 0)),
        ],
        out_specs=[
            pl.BlockSpec((_B, 1), lambda b: (0, 0)),
            pl.BlockSpec((_B, 1), lambda b: (0, 0)),
        ],
        out_shape=[
            jax.ShapeDtypeStruct((_B, 1), jnp.float32),
            jax.ShapeDtypeStruct((_B, 1), jnp.float32),
        ],
    )(dist_t, pt_t, types, coef)

    return (energy, energy_per_atom)
